# baseline (speedup 1.0000x reference)
import functools

import jax
import jax.numpy as jnp
from jax import lax
from jax.experimental import pallas as pl
from jax.experimental.pallas import tpu as pltpu
from jax.experimental.pallas import tpu_sc as plsc

NUM_CORES = 2
NUM_SUBCORES = 16
NUM_WORKERS = NUM_CORES * NUM_SUBCORES
CHUNK = 640
JT = CHUNK // 128
LANES = 16


def _make_gather(batch: int, dim: int):
  assert batch % (NUM_WORKERS * CHUNK) == 0 and dim == 32
  b_per_w = batch // NUM_WORKERS
  n_chunks = b_per_w // CHUNK
  assert n_chunks % 2 == 0
  n_pairs = n_chunks // 2
  n_dim_tiles = dim // 8
  tile_cols = batch // 128 * 1024
  mesh = plsc.VectorSubcoreMesh(
      core_axis_name="c",
      subcore_axis_name="s",
      num_cores=NUM_CORES,
      num_subcores=NUM_SUBCORES,
  )

  @functools.partial(
      pl.kernel,
      out_type=jax.ShapeDtypeStruct((n_dim_tiles, tile_cols), jnp.float32),
      mesh=mesh,
      scratch_types=[
          pltpu.VMEM((b_per_w,), jnp.int32),
          pltpu.VMEM((CHUNK, dim), jnp.float32),
          pltpu.VMEM((CHUNK, dim), jnp.float32),
          pltpu.VMEM((n_dim_tiles, JT * 1024), jnp.float32),
          pltpu.VMEM((n_dim_tiles, JT * 1024), jnp.float32),
          pltpu.SemaphoreType.DMA,
          pltpu.SemaphoreType.DMA,
          pltpu.SemaphoreType.DMA,
      ],
      compiler_params=pltpu.CompilerParams(use_tc_tiling_on_sc=False,
                                           needs_layout_passes=False),
  )
  def gather_kernel(ids_hbm, w_hbm, out_hbm, idx_v, rows0, rows1, stage0,
                    stage1, g_sem, o_sem0, o_sem1):
    wid = lax.axis_index("s") * NUM_CORES + lax.axis_index("c")
    base = wid * b_per_w
    out_base = wid * (b_per_w // 128) * 1024

    pltpu.sync_copy(ids_hbm.at[pl.ds(base, b_per_w)], idx_v)

    iota = lax.iota(jnp.int32, LANES)

    def g_start(j, rows):
      pltpu.async_copy(w_hbm.at[idx_v.at[pl.ds(j * CHUNK, CHUNK)]], rows,
                       g_sem)

    def g_wait(rows):
      pltpu.make_async_copy(w_hbm.at[idx_v.at[pl.ds(0, CHUNK)]], rows,
                            g_sem).wait()

    def shuffle(rows, stage):
      def body(t, carry):
        rowbase = t * LANES
        row_idx = rowbase + iota
        stage_off = (t // 8) * 1024 + (t % 8) * LANES
        for i in range(n_dim_tiles):
          for d in range(8):
            col_idx = jnp.full((LANES,), 8 * i + d, jnp.int32)
            v = plsc.load_gather(rows, [row_idx, col_idx])
            stage[i, pl.ds(stage_off + d * 128, LANES)] = v
        return carry

      lax.fori_loop(0, CHUNK // LANES, body, 0)

    def o_start(k, stage, sem):
      for i in range(n_dim_tiles):
        pltpu.async_copy(
            stage.at[i],
            out_hbm.at[i, pl.ds(out_base + k * JT * 1024, JT * 1024)], sem)

    def o_wait(stage, sem):
      for i in range(n_dim_tiles):
        pltpu.make_async_copy(stage.at[0],
                              out_hbm.at[0, pl.ds(0, JT * 1024)], sem).wait()

    g_start(0, rows0)

    def body(p, carry):
      i0 = 2 * p
      g_wait(rows0)
      g_start(i0 + 1, rows1)

      @pl.when(p > 0)
      def _():
        o_wait(stage0, o_sem0)

      shuffle(rows0, stage0)
      o_start(i0, stage0, o_sem0)

      g_wait(rows1)

      @pl.when(p + 1 < n_pairs)
      def _():
        g_start(i0 + 2, rows0)

      @pl.when(p > 0)
      def _():
        o_wait(stage1, o_sem1)

      shuffle(rows1, stage1)
      o_start(i0 + 1, stage1, o_sem1)
      return carry

    lax.fori_loop(0, n_pairs, body, 0)
    o_wait(stage0, o_sem0)
    o_wait(stage1, o_sem1)

  return gather_kernel


def kernel(camera_ids, weight):
  ids = camera_ids.reshape(-1).astype(jnp.int32)
  batch = ids.shape[0]
  dim = weight.shape[1]
  out2d = _make_gather(batch, dim)(ids, weight)
  out4d = out2d.reshape(dim // 8, batch // 128, 8, 128)
  return out4d.transpose((1, 3, 0, 2)).reshape(batch, dim)


# speedup vs baseline: 1.2793x; 1.2793x over previous
import functools

import jax
import jax.numpy as jnp
from jax import lax
from jax.experimental import pallas as pl
from jax.experimental.pallas import tpu as pltpu
from jax.experimental.pallas import tpu_sc as plsc

NUM_CORES = 2
NUM_SUBCORES = 16
NUM_WORKERS = NUM_CORES * NUM_SUBCORES
CHUNK = 640
JT = CHUNK // 128
LANES = 16


def _make_gather(batch: int, dim: int):
  assert batch % (NUM_WORKERS * CHUNK) == 0 and dim == 32
  b_per_w = batch // NUM_WORKERS
  n_chunks = b_per_w // CHUNK
  assert n_chunks % 2 == 0
  n_pairs = n_chunks // 2
  n_dim_tiles = dim // 8
  tile_cols = batch // 128 * 1024
  mesh = plsc.VectorSubcoreMesh(
      core_axis_name="c",
      subcore_axis_name="s",
      num_cores=NUM_CORES,
      num_subcores=NUM_SUBCORES,
  )

  @functools.partial(
      pl.kernel,
      out_type=jax.ShapeDtypeStruct((n_dim_tiles, tile_cols), jnp.float32),
      mesh=mesh,
      scratch_types=[
          pltpu.VMEM((b_per_w,), jnp.int32),
          pltpu.VMEM((CHUNK, dim), jnp.float32),
          pltpu.VMEM((CHUNK, dim), jnp.float32),
          pltpu.VMEM((n_dim_tiles, JT * 1024), jnp.float32),
          pltpu.VMEM((n_dim_tiles, JT * 1024), jnp.float32),
          pltpu.SemaphoreType.DMA,
          pltpu.SemaphoreType.DMA,
          pltpu.SemaphoreType.DMA,
      ],
      compiler_params=pltpu.CompilerParams(use_tc_tiling_on_sc=False,
                                           needs_layout_passes=False,
                                           disable_bounds_checks=True),
  )
  def gather_kernel(ids_hbm, w_hbm, out_hbm, idx_v, rows0, rows1, stage0,
                    stage1, g_sem, o_sem0, o_sem1):
    wid = lax.axis_index("s") * NUM_CORES + lax.axis_index("c")
    base = wid * b_per_w
    out_base = wid * (b_per_w // 128) * 1024

    pltpu.sync_copy(ids_hbm.at[pl.ds(base, b_per_w)], idx_v)

    iota = lax.iota(jnp.int32, LANES)

    def g_start(j, rows):
      pltpu.async_copy(w_hbm.at[idx_v.at[pl.ds(j * CHUNK, CHUNK)]], rows,
                       g_sem)

    def g_wait(rows):
      pltpu.make_async_copy(w_hbm.at[idx_v.at[pl.ds(0, CHUNK)]], rows,
                            g_sem).wait()

    def shuffle(rows, stage):
      def body(t, carry):
        rowbase = t * LANES
        row_idx = rowbase + iota
        stage_off = (t // 8) * 1024 + (t % 8) * LANES
        for i in range(n_dim_tiles):
          vs = []
          for d in range(8):
            col_idx = jnp.full((LANES,), 8 * i + d, jnp.int32)
            vs.append(plsc.load_gather(rows, [row_idx, col_idx]))
          for d in range(8):
            stage[i, pl.ds(stage_off + d * 128, LANES)] = vs[d]
        return carry

      lax.fori_loop(0, CHUNK // LANES, body, 0)

    def o_start(k, stage, sem):
      for i in range(n_dim_tiles):
        pltpu.async_copy(
            stage.at[i],
            out_hbm.at[i, pl.ds(out_base + k * JT * 1024, JT * 1024)], sem)

    def o_wait(stage, sem):
      for i in range(n_dim_tiles):
        pltpu.make_async_copy(stage.at[0],
                              out_hbm.at[0, pl.ds(0, JT * 1024)], sem).wait()

    g_start(0, rows0)

    def body(p, carry):
      i0 = 2 * p
      g_wait(rows0)
      g_start(i0 + 1, rows1)

      @pl.when(p > 0)
      def _():
        o_wait(stage0, o_sem0)

      shuffle(rows0, stage0)
      o_start(i0, stage0, o_sem0)

      g_wait(rows1)

      @pl.when(p + 1 < n_pairs)
      def _():
        g_start(i0 + 2, rows0)

      @pl.when(p > 0)
      def _():
        o_wait(stage1, o_sem1)

      shuffle(rows1, stage1)
      o_start(i0 + 1, stage1, o_sem1)
      return carry

    lax.fori_loop(0, n_pairs, body, 0)
    o_wait(stage0, o_sem0)
    o_wait(stage1, o_sem1)

  return gather_kernel


def kernel(camera_ids, weight):
  ids = camera_ids.reshape(-1).astype(jnp.int32)
  batch = ids.shape[0]
  dim = weight.shape[1]
  out2d = _make_gather(batch, dim)(ids, weight)
  out4d = out2d.reshape(dim // 8, batch // 128, 8, 128)
  return out4d.transpose((1, 3, 0, 2)).reshape(batch, dim)


# speedup vs baseline: 1.2823x; 1.0023x over previous
import functools

import jax
import jax.numpy as jnp
from jax import lax
from jax.experimental import pallas as pl
from jax.experimental.pallas import tpu as pltpu
from jax.experimental.pallas import tpu_sc as plsc

NUM_CORES = 2
NUM_SUBCORES = 16
NUM_WORKERS = NUM_CORES * NUM_SUBCORES
CHUNK = 640
JT = CHUNK // 128
LANES = 16


def _make_gather(batch: int, dim: int):
  assert batch % (NUM_WORKERS * CHUNK) == 0 and dim == 32
  b_per_w = batch // NUM_WORKERS
  n_chunks = b_per_w // CHUNK
  assert n_chunks % 2 == 0
  n_pairs = n_chunks // 2
  n_dim_tiles = dim // 8
  tile_cols = batch // 128 * 1024
  mesh = plsc.VectorSubcoreMesh(
      core_axis_name="c",
      subcore_axis_name="s",
      num_cores=NUM_CORES,
      num_subcores=NUM_SUBCORES,
  )

  @functools.partial(
      pl.kernel,
      out_type=jax.ShapeDtypeStruct((n_dim_tiles, tile_cols), jnp.float32),
      mesh=mesh,
      scratch_types=[
          pltpu.VMEM((b_per_w,), jnp.int32),
          pltpu.VMEM((CHUNK, dim), jnp.float32),
          pltpu.VMEM((CHUNK, dim), jnp.float32),
          pltpu.VMEM((n_dim_tiles, JT * 1024), jnp.float32),
          pltpu.VMEM((n_dim_tiles, JT * 1024), jnp.float32),
          pltpu.SemaphoreType.DMA,
          pltpu.SemaphoreType.DMA,
          pltpu.SemaphoreType.DMA,
      ],
      compiler_params=pltpu.CompilerParams(use_tc_tiling_on_sc=False,
                                           needs_layout_passes=False,
                                           disable_bounds_checks=True),
  )
  def gather_kernel(ids_hbm, w_hbm, out_hbm, idx_v, rows0, rows1, stage0,
                    stage1, g_sem, o_sem0, o_sem1):
    wid = lax.axis_index("s") * NUM_CORES + lax.axis_index("c")
    base = wid * b_per_w
    out_base = wid * (b_per_w // 128) * 1024

    pltpu.sync_copy(ids_hbm.at[pl.ds(base, b_per_w)], idx_v)

    iota = lax.iota(jnp.int32, LANES)

    def g_start(j, rows):
      pltpu.async_copy(w_hbm.at[idx_v.at[pl.ds(j * CHUNK, CHUNK)]], rows,
                       g_sem)

    def g_wait(rows):
      pltpu.make_async_copy(w_hbm.at[idx_v.at[pl.ds(0, CHUNK)]], rows,
                            g_sem).wait()

    def shuffle(rows, stage):
      @plsc.parallel_loop(0, CHUNK // LANES, step=1, unroll=4)
      def body(t):
        rowbase = t * LANES
        row_idx = rowbase + iota
        stage_off = (t // 8) * 1024 + (t % 8) * LANES
        for i in range(n_dim_tiles):
          vs = []
          for d in range(8):
            col_idx = jnp.full((LANES,), 8 * i + d, jnp.int32)
            vs.append(plsc.load_gather(rows, [row_idx, col_idx]))
          for d in range(8):
            stage[i, pl.ds(stage_off + d * 128, LANES)] = vs[d]

    def o_start(k, stage, sem):
      for i in range(n_dim_tiles):
        pltpu.async_copy(
            stage.at[i],
            out_hbm.at[i, pl.ds(out_base + k * JT * 1024, JT * 1024)], sem)

    def o_wait(stage, sem):
      for i in range(n_dim_tiles):
        pltpu.make_async_copy(stage.at[0],
                              out_hbm.at[0, pl.ds(0, JT * 1024)], sem).wait()

    g_start(0, rows0)

    def body(p, carry):
      i0 = 2 * p
      g_wait(rows0)
      g_start(i0 + 1, rows1)

      @pl.when(p > 0)
      def _():
        o_wait(stage0, o_sem0)

      shuffle(rows0, stage0)
      o_start(i0, stage0, o_sem0)

      g_wait(rows1)

      @pl.when(p + 1 < n_pairs)
      def _():
        g_start(i0 + 2, rows0)

      @pl.when(p > 0)
      def _():
        o_wait(stage1, o_sem1)

      shuffle(rows1, stage1)
      o_start(i0 + 1, stage1, o_sem1)
      return carry

    lax.fori_loop(0, n_pairs, body, 0)
    o_wait(stage0, o_sem0)
    o_wait(stage1, o_sem1)

  return gather_kernel


def kernel(camera_ids, weight):
  ids = camera_ids.reshape(-1).astype(jnp.int32)
  batch = ids.shape[0]
  dim = weight.shape[1]
  out2d = _make_gather(batch, dim)(ids, weight)
  out4d = out2d.reshape(dim // 8, batch // 128, 8, 128)
  return out4d.transpose((1, 3, 0, 2)).reshape(batch, dim)


# speedup vs baseline: 1.7269x; 1.3468x over previous
import functools

import jax
import jax.numpy as jnp
from jax import lax
from jax.experimental import pallas as pl
from jax.experimental.pallas import tpu as pltpu
from jax.experimental.pallas import tpu_sc as plsc

NUM_CORES = 2
NUM_SUBCORES = 16
NUM_WORKERS = NUM_CORES * NUM_SUBCORES
CHUNK = 640
JT = CHUNK // 128
LANES = 16


def _make_gather(batch: int, dim: int):
  assert batch % (NUM_WORKERS * CHUNK) == 0 and dim == 32
  b_per_w = batch // NUM_WORKERS
  n_chunks = b_per_w // CHUNK
  assert n_chunks % 2 == 0
  n_pairs = n_chunks // 2
  n_dim_tiles = dim // 8
  tile_cols = batch // 128 * 1024
  mesh = plsc.VectorSubcoreMesh(
      core_axis_name="c",
      subcore_axis_name="s",
      num_cores=NUM_CORES,
      num_subcores=NUM_SUBCORES,
  )

  @functools.partial(
      pl.kernel,
      out_type=jax.ShapeDtypeStruct((n_dim_tiles, tile_cols), jnp.float32),
      mesh=mesh,
      scratch_types=[
          pltpu.VMEM((b_per_w,), jnp.int32),
          pltpu.VMEM((CHUNK, dim), jnp.float32),
          pltpu.VMEM((CHUNK, dim), jnp.float32),
          pltpu.VMEM((n_dim_tiles * JT * 1024,), jnp.float32),
          pltpu.VMEM((n_dim_tiles * JT * 1024,), jnp.float32),
          pltpu.VMEM((2 * dim, LANES), jnp.int32),
          pltpu.SemaphoreType.DMA,
          pltpu.SemaphoreType.DMA,
          pltpu.SemaphoreType.DMA,
      ],
      compiler_params=pltpu.CompilerParams(use_tc_tiling_on_sc=False,
                                           needs_layout_passes=False,
                                           disable_bounds_checks=True),
  )
  def gather_kernel(ids_hbm, w_hbm, out_hbm, idx_v, rows0, rows1, stage0,
                    stage1, const_v, g_sem, o_sem0, o_sem1):
    wid = lax.axis_index("s") * NUM_CORES + lax.axis_index("c")
    base = wid * b_per_w
    out_base = wid * (b_per_w // 128) * 1024

    pltpu.sync_copy(ids_hbm.at[pl.ds(base, b_per_w)], idx_v)

    iota = lax.iota(jnp.int32, LANES)

    for dd in range(dim):
      c = (dd + iota) % dim
      const_v[2 * dd, pl.ds(0, LANES)] = c
      const_v[2 * dd + 1, pl.ds(0, LANES)] = (
          (c // 8) * (JT * 1024) + (c % 8) * 128 + iota)

    def g_start(j, rows):
      pltpu.async_copy(w_hbm.at[idx_v.at[pl.ds(j * CHUNK, CHUNK)]], rows,
                       g_sem)

    def g_wait(rows):
      pltpu.make_async_copy(w_hbm.at[idx_v.at[pl.ds(0, CHUNK)]], rows,
                            g_sem).wait()

    def shuffle(rows, stage):
      @plsc.parallel_loop(0, CHUNK // LANES, step=1, unroll=4)
      def body(t):
        row_idx = t * LANES + iota
        stage_off = (t // 8) * 1024 + (t % 8) * LANES
        for dd in range(dim):
          cload = const_v[2 * dd, pl.ds(0, LANES)]
          sflat = const_v[2 * dd + 1, pl.ds(0, LANES)]
          v = plsc.load_gather(rows, [row_idx, cload])
          plsc.store_scatter(stage, [sflat + stage_off], v)

    def o_start(k, stage, sem):
      for i in range(n_dim_tiles):
        pltpu.async_copy(
            stage.at[pl.ds(i * JT * 1024, JT * 1024)],
            out_hbm.at[i, pl.ds(out_base + k * JT * 1024, JT * 1024)], sem)

    def o_wait(stage, sem):
      for i in range(n_dim_tiles):
        pltpu.make_async_copy(stage.at[pl.ds(0, JT * 1024)],
                              out_hbm.at[0, pl.ds(0, JT * 1024)], sem).wait()

    g_start(0, rows0)

    def body(p, carry):
      i0 = 2 * p
      g_wait(rows0)
      g_start(i0 + 1, rows1)

      @pl.when(p > 0)
      def _():
        o_wait(stage0, o_sem0)

      shuffle(rows0, stage0)
      o_start(i0, stage0, o_sem0)

      g_wait(rows1)

      @pl.when(p + 1 < n_pairs)
      def _():
        g_start(i0 + 2, rows0)

      @pl.when(p > 0)
      def _():
        o_wait(stage1, o_sem1)

      shuffle(rows1, stage1)
      o_start(i0 + 1, stage1, o_sem1)
      return carry

    lax.fori_loop(0, n_pairs, body, 0)
    o_wait(stage0, o_sem0)
    o_wait(stage1, o_sem1)

  return gather_kernel


def kernel(camera_ids, weight):
  ids = camera_ids.reshape(-1).astype(jnp.int32)
  batch = ids.shape[0]
  dim = weight.shape[1]
  out2d = _make_gather(batch, dim)(ids, weight)
  out4d = out2d.reshape(dim // 8, batch // 128, 8, 128)
  return out4d.transpose((1, 3, 0, 2)).reshape(batch, dim)


# speedup vs baseline: 2.7977x; 1.6201x over previous
import functools

import jax
import jax.numpy as jnp
from jax import lax
from jax.experimental import pallas as pl
from jax.experimental.pallas import tpu as pltpu
from jax.experimental.pallas import tpu_sc as plsc

NUM_CORES = 2
NUM_SUBCORES = 16
NUM_WORKERS = NUM_CORES * NUM_SUBCORES
CHUNK = 640
JT = CHUNK // 128
LANES = 16


def _make_gather(batch: int, dim: int):
  assert batch % (NUM_WORKERS * CHUNK) == 0 and dim == 32
  b_per_w = batch // NUM_WORKERS
  n_chunks = b_per_w // CHUNK
  assert n_chunks % 2 == 0
  n_pairs = n_chunks // 2
  n_dim_tiles = dim // 8
  tile_cols = batch // 128 * 1024
  mesh = plsc.VectorSubcoreMesh(
      core_axis_name="c",
      subcore_axis_name="s",
      num_cores=NUM_CORES,
      num_subcores=NUM_SUBCORES,
  )

  @functools.partial(
      pl.kernel,
      out_type=jax.ShapeDtypeStruct((n_dim_tiles, tile_cols), jnp.float32),
      mesh=mesh,
      scratch_types=[
          pltpu.VMEM((b_per_w,), jnp.int32),
          pltpu.VMEM((CHUNK, dim), jnp.float32),
          pltpu.VMEM((CHUNK, dim), jnp.float32),
          pltpu.VMEM((n_dim_tiles * JT * 1024,), jnp.float32),
          pltpu.VMEM((n_dim_tiles * JT * 1024,), jnp.float32),
          pltpu.VMEM((2 * dim, LANES), jnp.int32),
          pltpu.SemaphoreType.DMA,
          pltpu.SemaphoreType.DMA,
          pltpu.SemaphoreType.DMA,
      ],
      compiler_params=pltpu.CompilerParams(use_tc_tiling_on_sc=False,
                                           needs_layout_passes=False,
                                           disable_bounds_checks=True),
  )
  def gather_kernel(ids_hbm, w_hbm, out_hbm, idx_v, rows0, rows1, stage0,
                    stage1, const_v, g_sem, o_sem0, o_sem1):
    wid = lax.axis_index("s") * NUM_CORES + lax.axis_index("c")
    base = wid * b_per_w
    out_base = wid * (b_per_w // 128) * 1024

    pltpu.sync_copy(ids_hbm.at[pl.ds(base, b_per_w)], idx_v)

    iota = lax.iota(jnp.int32, LANES)

    for dd in range(dim):
      c = (dd + iota) % dim
      const_v[2 * dd, pl.ds(0, LANES)] = c
      const_v[2 * dd + 1, pl.ds(0, LANES)] = (
          (c // 8) * (JT * 1024) + (c % 8) * 128 + iota)

    def g_start(j, rows):
      pltpu.async_copy(w_hbm.at[idx_v.at[pl.ds(j * CHUNK, CHUNK)]], rows,
                       g_sem)

    def g_wait(rows):
      pltpu.make_async_copy(w_hbm.at[idx_v.at[pl.ds(0, CHUNK)]], rows,
                            g_sem).wait()

    def shuffle(rows, stage):
      @plsc.parallel_loop(0, CHUNK // LANES, step=1, unroll=4)
      def body(t):
        row_idx = t * LANES + iota
        stage_off = (t // 8) * 1024 + (t % 8) * LANES
        for dd in range(dim):
          cload = const_v[2 * dd, pl.ds(0, LANES)]
          sflat = const_v[2 * dd + 1, pl.ds(0, LANES)]
          v = plsc.load_gather(rows, [row_idx, cload])
          plsc.store_scatter(stage, [sflat + stage_off], v)

    def o_start(k, stage, sem):
      for i in range(n_dim_tiles):
        pltpu.async_copy(
            stage.at[pl.ds(i * JT * 1024, JT * 1024)],
            out_hbm.at[i, pl.ds(out_base + k * JT * 1024, JT * 1024)], sem)

    def o_wait(stage, sem):
      for i in range(n_dim_tiles):
        pltpu.make_async_copy(stage.at[pl.ds(0, JT * 1024)],
                              out_hbm.at[0, pl.ds(0, JT * 1024)], sem).wait()

    g_start(0, rows0)

    def body(p, carry):
      i0 = 2 * p
      g_wait(rows0)
      g_start(i0 + 1, rows1)

      @pl.when(p > 0)
      def _():
        o_wait(stage0, o_sem0)

      shuffle(rows0, stage0)
      o_start(i0, stage0, o_sem0)

      g_wait(rows1)

      @pl.when(p + 1 < n_pairs)
      def _():
        g_start(i0 + 2, rows0)

      @pl.when(p > 0)
      def _():
        o_wait(stage1, o_sem1)

      shuffle(rows1, stage1)
      o_start(i0 + 1, stage1, o_sem1)
      return carry

    lax.fori_loop(0, n_pairs, body, 0)
    o_wait(stage0, o_sem0)
    o_wait(stage1, o_sem1)

  return gather_kernel


def _make_transpose(vocab: int, dim: int):
  full_tiles = vocab // 128
  per_w = full_tiles // NUM_WORKERS
  rem = full_tiles - per_w * NUM_WORKERS
  tail_rows = vocab - full_tiles * 128
  vp = full_tiles * 128 + 128
  assert per_w % 2 == 0 and 0 < tail_rows
  mesh = plsc.VectorSubcoreMesh(
      core_axis_name="c",
      subcore_axis_name="s",
      num_cores=NUM_CORES,
      num_subcores=NUM_SUBCORES,
  )

  @functools.partial(
      pl.kernel,
      out_type=jax.ShapeDtypeStruct((vp * dim,), jnp.float32),
      mesh=mesh,
      scratch_types=[
          pltpu.VMEM((dim, 128), jnp.float32),
          pltpu.VMEM((dim, 128), jnp.float32),
          pltpu.VMEM((128 * dim,), jnp.float32),
          pltpu.VMEM((128 * dim,), jnp.float32),
          pltpu.VMEM((tail_rows * dim,), jnp.float32),
          pltpu.VMEM((2 * dim, LANES), jnp.int32),
          pltpu.SemaphoreType.DMA,
          pltpu.SemaphoreType.DMA,
          pltpu.SemaphoreType.DMA,
      ],
      compiler_params=pltpu.CompilerParams(use_tc_tiling_on_sc=True,
                                           needs_layout_passes=False,
                                           disable_bounds_checks=True),
  )
  def transpose_kernel(wt_hbm, tail_hbm, out_hbm, vin0, vin1, vout0, vout1,
                       tbuf, const_v, i_sem, o_sem0, o_sem1):
    wid = lax.axis_index("s") * NUM_CORES + lax.axis_index("c")

    iota = lax.iota(jnp.int32, LANES)
    s32 = iota * dim
    for dd in range(dim):
      c = (dd + iota) % dim
      const_v[2 * dd, pl.ds(0, LANES)] = c
      const_v[2 * dd + 1, pl.ds(0, LANES)] = s32 + c

    def i_start(j, vin):
      for i in range(dim // 8):
        pltpu.async_copy(wt_hbm.at[pl.ds(8 * i, 8), pl.ds(j * 128, 128)],
                         vin.at[pl.ds(8 * i, 8), pl.ds(0, 128)], i_sem)

    def i_wait(vin):
      for i in range(dim // 8):
        pltpu.make_async_copy(wt_hbm.at[pl.ds(0, 8), pl.ds(0, 128)],
                              vin.at[pl.ds(0, 8), pl.ds(0, 128)],
                              i_sem).wait()

    def shuffle(vin, vout):
      @plsc.parallel_loop(0, 128 // LANES, step=1, unroll=4)
      def body(t):
        row_idx = t * LANES + iota
        v0x = t * LANES * dim
        for dd in range(dim):
          cload = const_v[2 * dd, pl.ds(0, LANES)]
          sflat = const_v[2 * dd + 1, pl.ds(0, LANES)]
          v = plsc.load_gather(vin, [cload, row_idx])
          plsc.store_scatter(vout, [sflat + v0x], v)

    def o_start(j, vout, sem):
      pltpu.async_copy(vout, out_hbm.at[pl.ds(j * 128 * dim, 128 * dim)],
                       sem)

    def o_wait(vout, sem):
      pltpu.make_async_copy(vout, out_hbm.at[pl.ds(0, 128 * dim)],
                            sem).wait()

    def jat(jj):
      return wid + NUM_WORKERS * jj

    i_start(jat(0), vin0)

    def body(p, carry):
      jj = 2 * p
      i_wait(vin0)
      i_start(jat(jj + 1), vin1)

      @pl.when(p > 0)
      def _():
        o_wait(vout0, o_sem0)

      shuffle(vin0, vout0)
      o_start(jat(jj), vout0, o_sem0)

      i_wait(vin1)

      @pl.when(p + 1 < per_w // 2)
      def _():
        i_start(jat(jj + 2), vin0)

      @pl.when(p > 0)
      def _():
        o_wait(vout1, o_sem1)

      shuffle(vin1, vout1)
      o_start(jat(jj + 1), vout1, o_sem1)
      return carry

    lax.fori_loop(0, per_w // 2, body, 0)
    o_wait(vout0, o_sem0)
    o_wait(vout1, o_sem1)

    @pl.when(wid < rem)
    def _():
      j = per_w * NUM_WORKERS + wid
      i_start(j, vin0)
      i_wait(vin0)
      shuffle(vin0, vout0)
      o_start(j, vout0, o_sem0)
      o_wait(vout0, o_sem0)

    @pl.when(wid == rem)
    def _():
      pltpu.sync_copy(tail_hbm, tbuf)
      pltpu.sync_copy(
          tbuf, out_hbm.at[pl.ds(full_tiles * 128 * dim, tail_rows * dim)])

  return transpose_kernel


def kernel(camera_ids, weight):
  ids = camera_ids.reshape(-1).astype(jnp.int32)
  batch = ids.shape[0]
  vocab, dim = weight.shape
  full_tiles = vocab // 128
  tail_lin = weight[full_tiles * 128:].reshape(-1)
  tout = _make_transpose(vocab, dim)(weight.T, tail_lin)
  w_rm = tout.reshape(full_tiles * 128 + 128, dim)
  out2d = _make_gather(batch, dim)(ids, w_rm)
  out4d = out2d.reshape(dim // 8, batch // 128, 8, 128)
  return out4d.transpose((1, 3, 0, 2)).reshape(batch, dim)


# speedup vs baseline: 3.0401x; 1.0867x over previous
import functools

import jax
import jax.numpy as jnp
from jax import lax
from jax.experimental import pallas as pl
from jax.experimental.pallas import tpu as pltpu
from jax.experimental.pallas import tpu_sc as plsc

NUM_CORES = 2
NUM_SUBCORES = 16
NUM_WORKERS = NUM_CORES * NUM_SUBCORES
CHUNK = 640
JT = CHUNK // 128
LANES = 16


def _make_gather(batch: int, dim: int):
  assert batch % (NUM_WORKERS * CHUNK) == 0 and dim == 32
  b_per_w = batch // NUM_WORKERS
  n_chunks = b_per_w // CHUNK
  assert n_chunks % 2 == 0
  n_pairs = n_chunks // 2
  n_dim_tiles = dim // 8
  tile_cols = batch // 128 * 1024
  mesh = plsc.VectorSubcoreMesh(
      core_axis_name="c",
      subcore_axis_name="s",
      num_cores=NUM_CORES,
      num_subcores=NUM_SUBCORES,
  )

  @functools.partial(
      pl.kernel,
      out_type=jax.ShapeDtypeStruct((n_dim_tiles, tile_cols), jnp.float32),
      mesh=mesh,
      scratch_types=[
          pltpu.VMEM((b_per_w,), jnp.int32),
          pltpu.VMEM((CHUNK, dim), jnp.float32),
          pltpu.VMEM((CHUNK, dim), jnp.float32),
          pltpu.VMEM((n_dim_tiles * JT * 1024,), jnp.float32),
          pltpu.VMEM((n_dim_tiles * JT * 1024,), jnp.float32),
          pltpu.VMEM((2 * dim, LANES), jnp.int32),
          pltpu.SemaphoreType.DMA,
          pltpu.SemaphoreType.DMA,
          pltpu.SemaphoreType.DMA,
      ],
      compiler_params=pltpu.CompilerParams(use_tc_tiling_on_sc=False,
                                           needs_layout_passes=False,
                                           disable_bounds_checks=True),
  )
  def gather_kernel(ids_hbm, w_hbm, out_hbm, idx_v, rows0, rows1, stage0,
                    stage1, const_v, g_sem, o_sem0, o_sem1):
    wid = lax.axis_index("s") * NUM_CORES + lax.axis_index("c")
    base = wid * b_per_w
    out_base = wid * (b_per_w // 128) * 1024

    pltpu.sync_copy(ids_hbm.at[pl.ds(base, b_per_w)], idx_v)

    iota = lax.iota(jnp.int32, LANES)

    for dd in range(dim):
      c = (dd + iota) % dim
      const_v[2 * dd, pl.ds(0, LANES)] = c
      const_v[2 * dd + 1, pl.ds(0, LANES)] = (
          (c // 8) * (JT * 1024) + (c % 8) * 128 + iota)

    def g_start(j, rows):
      pltpu.async_copy(w_hbm.at[idx_v.at[pl.ds(j * CHUNK, CHUNK)]], rows,
                       g_sem)

    def g_wait(rows):
      pltpu.make_async_copy(w_hbm.at[idx_v.at[pl.ds(0, CHUNK)]], rows,
                            g_sem).wait()

    def shuffle(rows, stage):
      @plsc.parallel_loop(0, CHUNK // LANES, step=1, unroll=4)
      def body(t):
        row_idx = t * LANES + iota
        stage_off = (t // 8) * 1024 + (t % 8) * LANES
        for dd in range(dim):
          cload = const_v[2 * dd, pl.ds(0, LANES)]
          sflat = const_v[2 * dd + 1, pl.ds(0, LANES)]
          v = plsc.load_gather(rows, [row_idx, cload])
          plsc.store_scatter(stage, [sflat + stage_off], v)

    def o_start(k, stage, sem):
      for i in range(n_dim_tiles):
        pltpu.async_copy(
            stage.at[pl.ds(i * JT * 1024, JT * 1024)],
            out_hbm.at[i, pl.ds(out_base + k * JT * 1024, JT * 1024)], sem)

    def o_wait(stage, sem):
      for i in range(n_dim_tiles):
        pltpu.make_async_copy(stage.at[pl.ds(0, JT * 1024)],
                              out_hbm.at[0, pl.ds(0, JT * 1024)], sem).wait()

    g_start(0, rows0)

    def body(p, carry):
      i0 = 2 * p
      g_wait(rows0)
      g_start(i0 + 1, rows1)

      @pl.when(p > 0)
      def _():
        o_wait(stage0, o_sem0)

      shuffle(rows0, stage0)
      o_start(i0, stage0, o_sem0)

      g_wait(rows1)

      @pl.when(p + 1 < n_pairs)
      def _():
        g_start(i0 + 2, rows0)

      @pl.when(p > 0)
      def _():
        o_wait(stage1, o_sem1)

      shuffle(rows1, stage1)
      o_start(i0 + 1, stage1, o_sem1)
      return carry

    lax.fori_loop(0, n_pairs, body, 0)
    o_wait(stage0, o_sem0)
    o_wait(stage1, o_sem1)

  return gather_kernel


def _make_transpose(vocab: int, dim: int):
  full_tiles = vocab // 128
  gsz = 4
  groups = full_tiles // (NUM_WORKERS * gsz)
  body_pairs = (groups - 1) // 2 if groups % 2 else groups // 2
  rem = full_tiles - groups * NUM_WORKERS * gsz
  tail_rows = vocab - full_tiles * 128
  vp = full_tiles * 128 + 128
  assert 0 < tail_rows and rem < NUM_WORKERS
  mesh = plsc.VectorSubcoreMesh(
      core_axis_name="c",
      subcore_axis_name="s",
      num_cores=NUM_CORES,
      num_subcores=NUM_SUBCORES,
  )

  @functools.partial(
      pl.kernel,
      out_type=jax.ShapeDtypeStruct((vp * dim,), jnp.float32),
      mesh=mesh,
      scratch_types=[
          pltpu.VMEM((dim, gsz * 128), jnp.float32),
          pltpu.VMEM((dim, gsz * 128), jnp.float32),
          pltpu.VMEM((gsz * 128 * dim,), jnp.float32),
          pltpu.VMEM((gsz * 128 * dim,), jnp.float32),
          pltpu.VMEM((tail_rows * dim,), jnp.float32),
          pltpu.VMEM((2 * dim, LANES), jnp.int32),
          pltpu.SemaphoreType.DMA,
          pltpu.SemaphoreType.DMA,
          pltpu.SemaphoreType.DMA,
      ],
      compiler_params=pltpu.CompilerParams(use_tc_tiling_on_sc=True,
                                           needs_layout_passes=False,
                                           disable_bounds_checks=True),
  )
  def transpose_kernel(wt_hbm, tail_hbm, out_hbm, vin0, vin1, vout0, vout1,
                       tbuf, const_v, i_sem, o_sem0, o_sem1):
    wid = lax.axis_index("s") * NUM_CORES + lax.axis_index("c")

    iota = lax.iota(jnp.int32, LANES)
    s32 = iota * dim
    for dd in range(dim):
      c = (dd + iota) % dim
      const_v[2 * dd, pl.ds(0, LANES)] = c
      const_v[2 * dd + 1, pl.ds(0, LANES)] = s32 + c

    gcols = gsz * 128

    def i_start(g, vin, ncols):
      for i in range(dim // 8):
        pltpu.async_copy(
            wt_hbm.at[pl.ds(8 * i, 8), pl.ds(g * gcols, ncols)],
            vin.at[pl.ds(8 * i, 8), pl.ds(0, ncols)], i_sem)

    def i_wait(vin, ncols):
      for i in range(dim // 8):
        pltpu.make_async_copy(wt_hbm.at[pl.ds(0, 8), pl.ds(0, ncols)],
                              vin.at[pl.ds(0, 8), pl.ds(0, ncols)],
                              i_sem).wait()

    def shuffle(vin, vout, ncols):
      @plsc.parallel_loop(0, ncols // LANES, step=1, unroll=4)
      def body(t):
        row_idx = t * LANES + iota
        v0x = t * LANES * dim
        for dd in range(dim):
          cload = const_v[2 * dd, pl.ds(0, LANES)]
          sflat = const_v[2 * dd + 1, pl.ds(0, LANES)]
          v = plsc.load_gather(vin, [cload, row_idx])
          plsc.store_scatter(vout, [sflat + v0x], v)

    def o_start(g, vout, sem):
      pltpu.async_copy(vout, out_hbm.at[pl.ds(g * gcols * dim, gcols * dim)],
                       sem)

    def o_wait(vout, sem):
      pltpu.make_async_copy(vout, out_hbm.at[pl.ds(0, gcols * dim)],
                            sem).wait()

    def gat(gg):
      return wid + NUM_WORKERS * gg

    i_start(gat(0), vin0, gcols)

    def body(p, carry):
      gg = 2 * p
      i_wait(vin0, gcols)
      i_start(gat(gg + 1), vin1, gcols)

      @pl.when(p > 0)
      def _():
        o_wait(vout0, o_sem0)

      shuffle(vin0, vout0, gcols)
      o_start(gat(gg), vout0, o_sem0)

      i_wait(vin1, gcols)

      @pl.when(p + 1 < body_pairs)
      def _():
        i_start(gat(gg + 2), vin0, gcols)

      @pl.when(p > 0)
      def _():
        o_wait(vout1, o_sem1)

      shuffle(vin1, vout1, gcols)
      o_start(gat(gg + 1), vout1, o_sem1)
      return carry

    lax.fori_loop(0, body_pairs, body, 0)
    o_wait(vout0, o_sem0)
    o_wait(vout1, o_sem1)

    if groups % 2:
      g_last = gat(groups - 1)
      i_start(g_last, vin0, gcols)
      i_wait(vin0, gcols)
      shuffle(vin0, vout0, gcols)
      o_start(g_last, vout0, o_sem0)
      o_wait(vout0, o_sem0)

    @pl.when(wid < rem)
    def _():
      jt = groups * NUM_WORKERS * gsz + wid
      for i in range(dim // 8):
        pltpu.sync_copy(wt_hbm.at[pl.ds(8 * i, 8), pl.ds(jt * 128, 128)],
                        vin1.at[pl.ds(8 * i, 8), pl.ds(0, 128)])
      shuffle(vin1, vout1, 128)
      pltpu.sync_copy(vout1.at[pl.ds(0, 128 * dim)],
                      out_hbm.at[pl.ds(jt * 128 * dim, 128 * dim)])

    @pl.when(wid == rem)
    def _():
      pltpu.sync_copy(tail_hbm, tbuf)
      pltpu.sync_copy(
          tbuf, out_hbm.at[pl.ds(full_tiles * 128 * dim, tail_rows * dim)])

  return transpose_kernel


def kernel(camera_ids, weight):
  ids = camera_ids.reshape(-1).astype(jnp.int32)
  batch = ids.shape[0]
  vocab, dim = weight.shape
  full_tiles = vocab // 128
  tail_lin = weight[full_tiles * 128:].reshape(-1)
  tout = _make_transpose(vocab, dim)(weight.T, tail_lin)
  w_rm = tout.reshape(full_tiles * 128 + 128, dim)
  out2d = _make_gather(batch, dim)(ids, w_rm)
  out4d = out2d.reshape(dim // 8, batch // 128, 8, 128)
  return out4d.transpose((1, 3, 0, 2)).reshape(batch, dim)
